# Initial kernel scaffold; baseline (speedup 1.0000x reference)
import functools

import jax
import jax.numpy as jnp
from jax import lax
from jax.experimental import pallas as pl
from jax.experimental.pallas import tpu as pltpu
from jax.experimental.pallas import tpu_sc as plsc

N = 10000
D = 128
E = 320000
G = 64
NCORES = 2
NSUB = 16
NW = NCORES * NSUB
K = 128
CH = 79
EPAD = NW * CH * K
NPAD = 10016
RPT = NPAD // NSUB
BN_EPS_K = 1e-5


def _sc_agg_body(h_hbm, col_hbm, row_hbm, zeros_hbm, out_hbm,
                 idxc_v, idxr_v, rows_v, accum_sh):
    c = lax.axis_index("c")
    s = lax.axis_index("s")
    wid = c * NSUB + s
    pltpu.sync_copy(zeros_hbm.at[pl.ds(s * RPT, RPT)],
                    accum_sh.at[pl.ds(s * RPT, RPT)])
    pltpu.sync_copy(col_hbm.at[wid], idxc_v)
    pltpu.sync_copy(row_hbm.at[wid], idxr_v)
    plsc.subcore_barrier()

    def chunk(j, carry):
        pltpu.sync_copy(h_hbm.at[idxc_v.at[j]], rows_v)
        pltpu.sync_copy(rows_v, accum_sh.at[idxr_v.at[j]], add=True)
        return carry

    lax.fori_loop(0, CH, chunk, 0)
    plsc.subcore_barrier()
    pltpu.sync_copy(accum_sh.at[pl.ds(s * RPT, RPT)],
                    out_hbm.at[c].at[pl.ds(s * RPT, RPT)])


_sc_agg = pl.kernel(
    _sc_agg_body,
    out_type=jax.ShapeDtypeStruct((NCORES, NPAD, D), jnp.float32),
    mesh=plsc.VectorSubcoreMesh(core_axis_name="c", subcore_axis_name="s"),
    scratch_types=[
        pltpu.MemorySpace.VMEM((CH, K), jnp.int32),
        pltpu.MemorySpace.VMEM((CH, K), jnp.int32),
        pltpu.MemorySpace.VMEM((K, D), jnp.float32),
        pltpu.MemorySpace.VMEM_SHARED((NPAD, D), jnp.float32),
    ],
)


def _bn_relu(z, gamma, beta):
    mean = jnp.mean(z, axis=0, keepdims=True)
    var = jnp.mean((z - mean) * (z - mean), axis=0, keepdims=True)
    zn = gamma * (z - mean) * lax.rsqrt(var + BN_EPS_K) + beta
    return jnp.maximum(zn, 0.0)


def _tc_layer_body(eps_ref, parts_ref, h_ref,
                   w0_ref, b0_ref, g0_ref, be0_ref,
                   w1_ref, b1_ref, gl_ref, bel_ref, out_ref):
    eps = eps_ref[0]
    pooled = (parts_ref[0, 0:N, :] + parts_ref[1, 0:N, :]
              + (1.0 + eps) * h_ref[...])
    z = lax.dot_general(pooled, w0_ref[...], (((1,), (1,)), ((), ())),
                        preferred_element_type=jnp.float32) + b0_ref[...]
    z = _bn_relu(z, g0_ref[...], be0_ref[...])
    z = lax.dot_general(z, w1_ref[...], (((1,), (1,)), ((), ())),
                        preferred_element_type=jnp.float32) + b1_ref[...]
    out_ref[...] = _bn_relu(z, gl_ref[...], bel_ref[...])


_tc_layer = pl.pallas_call(
    _tc_layer_body,
    out_shape=jax.ShapeDtypeStruct((N, D), jnp.float32),
    in_specs=[
        pl.BlockSpec(memory_space=pltpu.MemorySpace.SMEM),
    ] + [pl.BlockSpec(memory_space=pltpu.MemorySpace.VMEM)] * 10,
    out_specs=pl.BlockSpec(memory_space=pltpu.MemorySpace.VMEM),
)


def _tc_pool_body(batch_ref, h0_ref, h1_ref, h2_ref, h3_ref, h4_ref,
                  w_ref, b_ref, out_ref):
    b = jnp.broadcast_to(batch_ref[...], (G, N))
    gi = lax.broadcasted_iota(jnp.int32, (G, N), 0)
    p = (b == gi).astype(jnp.float32)
    counts = jnp.sum(p, axis=1, keepdims=True)
    inv = 1.0 / jnp.maximum(counts, 1.0)
    acc = jnp.zeros((G, D), jnp.float32)
    for l in range(5):
        h_ref = (h0_ref, h1_ref, h2_ref, h3_ref, h4_ref)[l]
        pooled = lax.dot_general(p, h_ref[...], (((1,), (0,)), ((), ())),
                                 preferred_element_type=jnp.float32) * inv
        acc = acc + lax.dot_general(pooled, w_ref[l],
                                    (((1,), (1,)), ((), ())),
                                    preferred_element_type=jnp.float32)
        acc = acc + b_ref[l]
    out_ref[...] = acc


_tc_pool = pl.pallas_call(
    _tc_pool_body,
    out_shape=jax.ShapeDtypeStruct((G, D), jnp.float32),
)


def kernel(x, params, edge_index, batch):
    row = edge_index[0]
    col = edge_index[1]
    pad = EPAD - E
    colp = jnp.concatenate([col, jnp.zeros((pad,), jnp.int32)]).reshape(NW, CH, K)
    rowp = jnp.concatenate([row, jnp.full((pad,), N, jnp.int32)]).reshape(NW, CH, K)
    zeros_init = jnp.zeros((NPAD, D), jnp.float32)

    hs = [x]
    h = x
    for l in range(4):
        parts = _sc_agg(h, colp, rowp, zeros_init)
        mlp = params["mlp%d" % l]
        h = _tc_layer(
            params["eps"][l].reshape(1),
            parts, h,
            mlp["W0"], mlp["b0"].reshape(1, D),
            mlp["bn_g0"].reshape(1, D), mlp["bn_b0"].reshape(1, D),
            mlp["W1"], mlp["b1"].reshape(1, D),
            params["bn_g%d" % l].reshape(1, D),
            params["bn_b%d" % l].reshape(1, D),
        )
        hs.append(h)

    wstack = jnp.stack([params["pred%d_W" % l] for l in range(5)])
    bstack = jnp.stack([params["pred%d_b" % l] for l in range(5)]).reshape(5, 1, D)
    score = _tc_pool(batch.reshape(1, N), *hs, wstack, bstack)
    return score


# speedup vs baseline: 4.0814x; 4.0814x over previous
import functools

import jax
import jax.numpy as jnp
from jax import lax
from jax.experimental import pallas as pl
from jax.experimental.pallas import tpu as pltpu
from jax.experimental.pallas import tpu_sc as plsc

N = 10000
D = 128
E = 320000
G = 64
NCORES = 2
NSUB = 16
NW = NCORES * NSUB
K = 128
CH = 79
EPAD = NW * CH * K
NPAD = 10112
RPT = NPAD // NSUB
BN_EPS_K = 1e-5


def _sc_agg_body(h_hbm, col_hbm, row_hbm, zeros_hbm, out_hbm,
                 idxc_v, idxr_v, rows_v, accum_sh):
    c = lax.axis_index("c")
    s = lax.axis_index("s")
    wid = c * NSUB + s
    pltpu.sync_copy(zeros_hbm.at[pl.ds(s * RPT, RPT)],
                    accum_sh.at[pl.ds(s * RPT, RPT)])
    pltpu.sync_copy(col_hbm.at[wid], idxc_v)
    pltpu.sync_copy(row_hbm.at[wid], idxr_v)
    plsc.subcore_barrier()

    def chunk(j, carry):
        pltpu.sync_copy(h_hbm.at[idxc_v.at[j]], rows_v)
        pltpu.sync_copy(rows_v, accum_sh.at[idxr_v.at[j]], add=True)
        return carry

    lax.fori_loop(0, CH, chunk, 0)
    plsc.subcore_barrier()
    pltpu.sync_copy(accum_sh.at[pl.ds(s * RPT, RPT)],
                    out_hbm.at[c].at[pl.ds(s * RPT, RPT)])


_sc_agg = pl.kernel(
    _sc_agg_body,
    out_type=jax.ShapeDtypeStruct((NCORES, NPAD, D), jnp.float32),
    mesh=plsc.VectorSubcoreMesh(core_axis_name="c", subcore_axis_name="s"),
    scratch_types=[
        pltpu.MemorySpace.VMEM((CH, K), jnp.int32),
        pltpu.MemorySpace.VMEM((CH, K), jnp.int32),
        pltpu.MemorySpace.VMEM((K, D), jnp.float32),
        pltpu.MemorySpace.VMEM_SHARED((NPAD, D), jnp.float32),
    ],
)


def _bn_relu(z, gamma, beta):
    mean = jnp.mean(z, axis=0, keepdims=True)
    var = jnp.mean((z - mean) * (z - mean), axis=0, keepdims=True)
    zn = gamma * (z - mean) * lax.rsqrt(var + BN_EPS_K) + beta
    return jnp.maximum(zn, 0.0)


def _tc_layer_body(eps_ref, parts_ref, h_ref,
                   w0_ref, b0_ref, g0_ref, be0_ref,
                   w1_ref, b1_ref, gl_ref, bel_ref, out_ref):
    eps = eps_ref[0]
    pooled = (parts_ref[0, 0:N, :] + parts_ref[1, 0:N, :]
              + (1.0 + eps) * h_ref[...])
    z = lax.dot_general(pooled, w0_ref[...], (((1,), (1,)), ((), ())),
                        preferred_element_type=jnp.float32) + b0_ref[...]
    z = _bn_relu(z, g0_ref[...], be0_ref[...])
    z = lax.dot_general(z, w1_ref[...], (((1,), (1,)), ((), ())),
                        preferred_element_type=jnp.float32) + b1_ref[...]
    out_ref[...] = _bn_relu(z, gl_ref[...], bel_ref[...])


_tc_layer = pl.pallas_call(
    _tc_layer_body,
    out_shape=jax.ShapeDtypeStruct((N, D), jnp.float32),
    in_specs=[
        pl.BlockSpec(memory_space=pltpu.MemorySpace.SMEM),
    ] + [pl.BlockSpec(memory_space=pltpu.MemorySpace.VMEM)] * 10,
    out_specs=pl.BlockSpec(memory_space=pltpu.MemorySpace.VMEM),
)


def _tc_pool_body(batch_ref, h0_ref, h1_ref, h2_ref, h3_ref, h4_ref,
                  w_ref, b_ref, out_ref):
    b = jnp.broadcast_to(batch_ref[...], (G, N))
    gi = lax.broadcasted_iota(jnp.int32, (G, N), 0)
    p = (b == gi).astype(jnp.float32)
    counts = jnp.sum(p, axis=1, keepdims=True)
    inv = 1.0 / jnp.maximum(counts, 1.0)
    acc = jnp.zeros((G, D), jnp.float32)
    for l in range(5):
        h_ref = (h0_ref, h1_ref, h2_ref, h3_ref, h4_ref)[l]
        pooled = lax.dot_general(p, h_ref[...], (((1,), (0,)), ((), ())),
                                 preferred_element_type=jnp.float32) * inv
        acc = acc + lax.dot_general(pooled, w_ref[l],
                                    (((1,), (1,)), ((), ())),
                                    preferred_element_type=jnp.float32)
        acc = acc + b_ref[l]
    out_ref[...] = acc


_tc_pool = pl.pallas_call(
    _tc_pool_body,
    out_shape=jax.ShapeDtypeStruct((G, D), jnp.float32),
)


def kernel(x, params, edge_index, batch):
    row = edge_index[0]
    col = edge_index[1]
    pad = EPAD - E
    colp = jnp.concatenate([col, jnp.zeros((pad,), jnp.int32)]).reshape(NW, CH, K)
    rowp = jnp.concatenate([row, jnp.full((pad,), N, jnp.int32)]).reshape(NW, CH, K)
    zeros_init = jnp.zeros((NPAD, D), jnp.float32)

    hs = [x]
    h = x
    for l in range(4):
        parts = _sc_agg(h, colp, rowp, zeros_init)
        mlp = params["mlp%d" % l]
        h = _tc_layer(
            params["eps"][l].reshape(1),
            parts, h,
            mlp["W0"], mlp["b0"].reshape(1, D),
            mlp["bn_g0"].reshape(1, D), mlp["bn_b0"].reshape(1, D),
            mlp["W1"], mlp["b1"].reshape(1, D),
            params["bn_g%d" % l].reshape(1, D),
            params["bn_b%d" % l].reshape(1, D),
        )
        hs.append(h)

    wstack = jnp.stack([params["pred%d_W" % l] for l in range(5)])
    bstack = jnp.stack([params["pred%d_b" % l] for l in range(5)]).reshape(5, 1, D)
    score = _tc_pool(batch.reshape(1, N), *hs, wstack, bstack)
    return score
